# baseline (speedup 1.0000x reference)
import functools

import jax
import jax.numpy as jnp
from jax import lax
from jax.experimental import pallas as pl
from jax.experimental.pallas import tpu as pltpu
from jax.experimental.pallas import tpu_sc as plsc

N_HID = 128
N = 320000
BLK = 80
NBLK = N // BLK
NC, NS = 2, 16
NW = NC * NS
BLOCKS_PER_W = NBLK // NW


def _proj_body(emb_ref, w_ref, b_ref, out_ref):
    out_ref[...] = lax.dot_general(
        emb_ref[...], w_ref[...],
        (((1,), (1,)), ((), ())),
        preferred_element_type=jnp.float32,
    ) + b_ref[...]


NBUF = 5
GROUPS = BLOCKS_PER_W // NBUF


def _sc_body(ptable_hbm, t_hbm, out_hbm, idx_v, tbl_v, rows_v, gsem, wsem):
    w = lax.axis_index("s") * NC + lax.axis_index("c")
    base = w * BLOCKS_PER_W
    pltpu.sync_copy(t_hbm.at[w], idx_v)

    @pl.when(lax.axis_index("s") == 0)
    def _():
        pltpu.sync_copy(ptable_hbm, tbl_v)

    plsc.subcore_barrier()

    def group(i, carry):
        gbase = base + i * NBUF

        @pl.when(i > 0)
        def _():
            for b in range(NBUF):
                pltpu.make_async_copy(
                    out_hbm.at[pl.ds(0, BLK)], rows_v.at[b], wsem.at[b]
                ).wait()

        descs = [
            pltpu.async_copy(
                ptable_hbm.at[idx_v.at[i * NBUF + b]], rows_v.at[b], gsem.at[b]
            )
            for b in range(NBUF)
        ]
        for b in range(NBUF):
            descs[b].wait()
            pltpu.async_copy(
                rows_v.at[b],
                out_hbm.at[pl.ds((gbase + b) * BLK, BLK)],
                wsem.at[b],
            )
        return carry

    lax.fori_loop(0, GROUPS, group, 0)
    for b in range(NBUF):
        pltpu.make_async_copy(
            out_hbm.at[pl.ds(0, BLK)], rows_v.at[b], wsem.at[b]
        ).wait()


_mesh = plsc.VectorSubcoreMesh(
    core_axis_name="c", subcore_axis_name="s", num_cores=NC, num_subcores=NS
)

_sc_gather = functools.partial(
    pl.kernel,
    mesh=_mesh,
    out_type=jax.ShapeDtypeStruct((N, N_HID), jnp.float32),
    scratch_types=[
        pltpu.VMEM((BLOCKS_PER_W, BLK), jnp.int32),
        pltpu.VMEM_SHARED((8, N_HID), jnp.float32),
        pltpu.VMEM((NBUF, BLK, N_HID), jnp.float32),
        pltpu.SemaphoreType.DMA((NBUF,)),
        pltpu.SemaphoreType.DMA((NBUF,)),
    ],
)(_sc_body)


def kernel(t, emb_table, W, b):
    emb8 = jnp.concatenate(
        [emb_table, jnp.zeros((1, N_HID), jnp.float32)], axis=0
    )
    ptable = pl.pallas_call(
        _proj_body,
        out_shape=jax.ShapeDtypeStruct((8, N_HID), jnp.float32),
    )(emb8, W, b.reshape(1, N_HID))
    return _sc_gather(ptable, t.reshape(NW, BLOCKS_PER_W, BLK))


# speedup vs baseline: 20.9524x; 20.9524x over previous
import functools

import jax
import jax.numpy as jnp
from jax import lax
from jax.experimental import pallas as pl
from jax.experimental.pallas import tpu as pltpu
from jax.experimental.pallas import tpu_sc as plsc

N_HID = 128
N = 320000
BLK = 40
NBLK = N // BLK
NC, NS = 2, 16
NW = NC * NS
BLOCKS_PER_W = NBLK // NW


def _proj_body(emb_ref, w_ref, b_ref, out_ref):
    out_ref[...] = lax.dot_general(
        emb_ref[...], w_ref[...],
        (((1,), (1,)), ((), ())),
        preferred_element_type=jnp.float32,
    ) + b_ref[...]


NBUF = 10
GROUPS = BLOCKS_PER_W // NBUF


def _sc_body(ptable_hbm, t_hbm, out_hbm, idx_v, tbl_v, rows_v, gsem, wsem):
    w = lax.axis_index("s") * NC + lax.axis_index("c")
    base = w * BLOCKS_PER_W
    pltpu.sync_copy(t_hbm.at[w], idx_v)

    @pl.when(lax.axis_index("s") == 0)
    def _():
        pltpu.sync_copy(ptable_hbm, tbl_v)

    plsc.subcore_barrier()

    def group(i, carry):
        gbase = base + i * NBUF

        @pl.when(i > 0)
        def _():
            for b in range(NBUF):
                pltpu.make_async_copy(
                    out_hbm.at[pl.ds(0, BLK)], rows_v.at[b], wsem.at[b]
                ).wait()

        descs = [
            pltpu.async_copy(
                tbl_v.at[idx_v.at[i * NBUF + b]], rows_v.at[b], gsem.at[b]
            )
            for b in range(NBUF)
        ]
        for b in range(NBUF):
            descs[b].wait()
            pltpu.async_copy(
                rows_v.at[b],
                out_hbm.at[pl.ds((gbase + b) * BLK, BLK)],
                wsem.at[b],
            )
        return carry

    lax.fori_loop(0, GROUPS, group, 0)
    for b in range(NBUF):
        pltpu.make_async_copy(
            out_hbm.at[pl.ds(0, BLK)], rows_v.at[b], wsem.at[b]
        ).wait()


_mesh = plsc.VectorSubcoreMesh(
    core_axis_name="c", subcore_axis_name="s", num_cores=NC, num_subcores=NS
)

_sc_gather = functools.partial(
    pl.kernel,
    mesh=_mesh,
    out_type=jax.ShapeDtypeStruct((N, N_HID), jnp.float32),
    scratch_types=[
        pltpu.VMEM((BLOCKS_PER_W, BLK), jnp.int32),
        pltpu.VMEM_SHARED((8, N_HID), jnp.float32),
        pltpu.VMEM((NBUF, BLK, N_HID), jnp.float32),
        pltpu.SemaphoreType.DMA((NBUF,)),
        pltpu.SemaphoreType.DMA((NBUF,)),
    ],
)(_sc_body)


def kernel(t, emb_table, W, b):
    emb8 = jnp.concatenate(
        [emb_table, jnp.zeros((1, N_HID), jnp.float32)], axis=0
    )
    ptable = pl.pallas_call(
        _proj_body,
        out_shape=jax.ShapeDtypeStruct((8, N_HID), jnp.float32),
    )(emb8, W, b.reshape(1, N_HID))
    return _sc_gather(ptable, t.reshape(NW, BLOCKS_PER_W, BLK))
